# baseline (speedup 1.0000x reference)
import jax
import jax.numpy as jnp
from jax.experimental import pallas as pl
from jax.experimental.pallas import tpu as pltpu

_B, _T, _C, _H, _W = 1, 2, 128, 128, 128
_NH = 4
_HD = _C // _NH
_WS = 8
_K = 8
_PAD = _WS // 2
_HP = _H + 2 * _PAD
_WP = _W + 2 * _PAD
_SCALE = _HD ** (-0.5)

_RPROJ = 64
_RATT = 8

_HIGH = jax.lax.Precision.DEFAULT


def _qkv_body(x_ref, wq_ref, bq_ref, wk_ref, bk_ref, wv_ref, bv_ref,
              q_ref, k_ref, v_ref):
    x = x_ref[0].reshape(_C, _RPROJ * _W)

    def proj(w_ref, b_ref):
        y = jax.lax.dot_general(w_ref[...], x, (((0,), (0,)), ((), ())),
                                preferred_element_type=jnp.float32,
                                precision=_HIGH)
        return y + b_ref[...]

    q_ref[0] = (proj(wq_ref, bq_ref) * _SCALE).reshape(_C, _RPROJ, _W)
    k_ref[0] = proj(wk_ref, bk_ref).reshape(_C, _RPROJ, _W)
    v_ref[0] = proj(wv_ref, bv_ref).reshape(_C, _RPROJ, _W)


def _rev8(a, lane):
    for d in (4, 2, 1):
        a = jnp.where(lane % (2 * d) < d,
                      jnp.roll(a, -d, axis=-1), jnp.roll(a, d, axis=-1))
    return a


def _xshift(win, r8, lane, s):
    if s == 0:
        return win
    if s < 0:
        return jnp.where(lane < -s,
                         jnp.roll(r8, -(7 + s), axis=-1),
                         jnp.roll(win, -s, axis=-1))
    return jnp.where(lane >= _W - s,
                     jnp.roll(r8, 7 - s, axis=-1),
                     jnp.roll(win, -s, axis=-1))


def _attn_body(q_ref, kl_ref, kh_ref, vl_ref, vh_ref, o_ref):
    qh = q_ref[0]

    kwin = jnp.concatenate([kl_ref[0], kh_ref[0]], axis=1)
    lane = jax.lax.broadcasted_iota(jnp.int32, kwin.shape, 2)
    krev = _rev8(kwin, lane)
    ds = []
    for dxi in range(_WS):
        kx = _xshift(kwin, krev, lane, dxi - _PAD)
        for dyi in range(_WS):
            ds.append(jnp.sum(qh * kx[:, dyi:dyi + _RATT, :], axis=0))
    dists = jnp.stack(ds, axis=0)

    cur = dists
    m1 = jnp.max(dists, axis=0)
    coef = jnp.zeros(dists.shape, jnp.float32)
    slots = jnp.full((_RATT, _W), float(_K), jnp.float32)
    denom = jnp.zeros((_RATT, _W), jnp.float32)
    for _ in range(_K):
        m = jnp.max(cur, axis=0)
        eqm = cur == m[None]
        c = jnp.sum(eqm.astype(jnp.float32), axis=0)
        take = jnp.minimum(c, slots)
        e = jnp.exp(m - m1)
        denom = denom + e * take
        coef = jnp.where(eqm, (e * take / c)[None], coef)
        slots = slots - take
        cur = jnp.where(eqm, -jnp.inf, cur)

    vwin = jnp.concatenate([vl_ref[0], vh_ref[0]], axis=1)
    vrev = _rev8(vwin, lane)
    acc = jnp.zeros((_HD, _RATT, _W), jnp.float32)
    for dxi in range(_WS):
        vx = _xshift(vwin, vrev, lane, dxi - _PAD)
        for dyi in range(_WS):
            acc = acc + (coef[dxi * _WS + dyi][None]
                         * vx[:, dyi:dyi + _RATT, :])
    o_ref[0] = acc / denom[None]


def _out_body(a_ref, wp_ref, bp_ref, o_ref):
    a = a_ref[0].reshape(_C, _RPROJ * _W)
    y = jax.lax.dot_general(wp_ref[...], a, (((0,), (0,)), ((), ())),
                            preferred_element_type=jnp.float32,
                            precision=_HIGH)
    o_ref[0] = (y + bp_ref[...]).reshape(_C, _RPROJ, _W)


def _forward(vid, Wq, bq, Wk, bk, Wv, bv, Wp, bp, interpret=False):
    vid4 = vid[0]

    def col(b):
        return b.reshape(_C, 1)

    w_spec = pl.BlockSpec((_C, _C), lambda t, r: (0, 0))
    b_spec = pl.BlockSpec((_C, 1), lambda t, r: (0, 0))
    img_spec = pl.BlockSpec((1, _C, _RPROJ, _W), lambda t, r: (t, 0, r, 0))

    q, k, v = pl.pallas_call(
        _qkv_body,
        grid=(_T, _H // _RPROJ),
        in_specs=[img_spec, w_spec, b_spec, w_spec, b_spec, w_spec, b_spec],
        out_specs=[img_spec, img_spec, img_spec],
        out_shape=[jax.ShapeDtypeStruct((_T, _C, _H, _W), jnp.float32)] * 3,
        compiler_params=pltpu.CompilerParams(
            dimension_semantics=("parallel", "parallel")),
        interpret=interpret,
    )(vid4, Wq, col(bq), Wk, col(bk), Wv, col(bv))

    pad = ((0, 0), (0, 0), (_PAD, _PAD), (0, 0))
    kp = jnp.pad(k, pad, mode="reflect")
    vp = jnp.pad(v, pad, mode="reflect")

    blk = pl.BlockSpec((1, _HD, _RATT, _W), lambda t, h, r: (t, h, r, 0))
    blk_hi = pl.BlockSpec((1, _HD, _RATT, _W), lambda t, h, r: (t, h, r + 1, 0))
    attn = pl.pallas_call(
        _attn_body,
        grid=(_T, _NH, _H // _RATT),
        in_specs=[blk, blk, blk_hi, blk, blk_hi],
        out_specs=blk,
        out_shape=jax.ShapeDtypeStruct((_T, _C, _H, _W), jnp.float32),
        compiler_params=pltpu.CompilerParams(
            dimension_semantics=("parallel", "parallel", "parallel")),
        interpret=interpret,
    )(q, kp, kp, vp, vp)

    out = pl.pallas_call(
        _out_body,
        grid=(_T, _H // _RPROJ),
        in_specs=[img_spec, w_spec, b_spec],
        out_specs=img_spec,
        out_shape=jax.ShapeDtypeStruct((_T, _C, _H, _W), jnp.float32),
        compiler_params=pltpu.CompilerParams(
            dimension_semantics=("parallel", "parallel")),
        interpret=interpret,
    )(attn, Wp, col(bp))

    return out[None]


def kernel(vid, Wq, bq, Wk, bk, Wv, bv, Wp, bp):
    return _forward(vid, Wq, bq, Wk, bk, Wv, bv, Wp, bp)


# speedup vs baseline: 1.1039x; 1.1039x over previous
import jax
import jax.numpy as jnp
from jax.experimental import pallas as pl
from jax.experimental.pallas import tpu as pltpu

_B, _T, _C, _H, _W = 1, 2, 128, 128, 128
_NH = 4
_HD = _C // _NH
_WS = 8
_K = 8
_PAD = _WS // 2
_HP = _H + 2 * _PAD
_WP = _W + 2 * _PAD
_SCALE = _HD ** (-0.5)

_RPROJ = 64
_RATT = 16

_HIGH = jax.lax.Precision.DEFAULT


def _qkv_body(x_ref, wq_ref, bq_ref, wk_ref, bk_ref, wv_ref, bv_ref,
              q_ref, k_ref, v_ref):
    x = x_ref[0].reshape(_C, _RPROJ * _W)

    def proj(w_ref, b_ref):
        y = jax.lax.dot_general(w_ref[...], x, (((0,), (0,)), ((), ())),
                                preferred_element_type=jnp.float32,
                                precision=_HIGH)
        return y + b_ref[...]

    q_ref[0] = (proj(wq_ref, bq_ref) * _SCALE).reshape(_C, _RPROJ, _W)
    k_ref[0] = proj(wk_ref, bk_ref).reshape(_C, _RPROJ, _W)
    v_ref[0] = proj(wv_ref, bv_ref).reshape(_C, _RPROJ, _W)


def _rev8(a, lane):
    for d in (4, 2, 1):
        a = jnp.where(lane % (2 * d) < d,
                      jnp.roll(a, -d, axis=-1), jnp.roll(a, d, axis=-1))
    return a


def _xshift(win, r8, lane, s):
    if s == 0:
        return win
    if s < 0:
        return jnp.where(lane < -s,
                         jnp.roll(r8, -(7 + s), axis=-1),
                         jnp.roll(win, -s, axis=-1))
    return jnp.where(lane >= _W - s,
                     jnp.roll(r8, 7 - s, axis=-1),
                     jnp.roll(win, -s, axis=-1))


def _attn_body(q_ref, kl_ref, kh_ref, vl_ref, vh_ref, o_ref):
    qh = q_ref[0]

    kwin = jnp.concatenate([kl_ref[0], kh_ref[0]], axis=1)
    lane = jax.lax.broadcasted_iota(jnp.int32, kwin.shape, 2)
    krev = _rev8(kwin, lane)
    ds = []
    for dxi in range(_WS):
        kx = _xshift(kwin, krev, lane, dxi - _PAD)
        for dyi in range(_WS):
            ds.append(jnp.sum(qh * kx[:, dyi:dyi + _RATT, :], axis=0))
    dists = jnp.stack(ds, axis=0)

    cur = dists
    m1 = jnp.max(dists, axis=0)
    coef = jnp.zeros(dists.shape, jnp.float32)
    slots = jnp.full((_RATT, _W), float(_K), jnp.float32)
    denom = jnp.zeros((_RATT, _W), jnp.float32)
    for _ in range(_K):
        m = jnp.max(cur, axis=0)
        eqm = cur == m[None]
        c = jnp.sum(eqm.astype(jnp.float32), axis=0)
        take = jnp.minimum(c, slots)
        e = jnp.exp(m - m1)
        denom = denom + e * take
        coef = jnp.where(eqm, (e * take / c)[None], coef)
        slots = slots - take
        cur = jnp.where(eqm, -jnp.inf, cur)

    vwin = jnp.concatenate([vl_ref[0], vh_ref[0]], axis=1)
    vrev = _rev8(vwin, lane)
    acc = jnp.zeros((_HD, _RATT, _W), jnp.float32)
    for dxi in range(_WS):
        vx = _xshift(vwin, vrev, lane, dxi - _PAD)
        for dyi in range(_WS):
            acc = acc + (coef[dxi * _WS + dyi][None]
                         * vx[:, dyi:dyi + _RATT, :])
    o_ref[0] = acc / denom[None]


def _out_body(a_ref, wp_ref, bp_ref, o_ref):
    a = a_ref[0].reshape(_C, _RPROJ * _W)
    y = jax.lax.dot_general(wp_ref[...], a, (((0,), (0,)), ((), ())),
                            preferred_element_type=jnp.float32,
                            precision=_HIGH)
    o_ref[0] = (y + bp_ref[...]).reshape(_C, _RPROJ, _W)


def _forward(vid, Wq, bq, Wk, bk, Wv, bv, Wp, bp, interpret=False):
    vid4 = vid[0]

    def col(b):
        return b.reshape(_C, 1)

    w_spec = pl.BlockSpec((_C, _C), lambda t, r: (0, 0))
    b_spec = pl.BlockSpec((_C, 1), lambda t, r: (0, 0))
    img_spec = pl.BlockSpec((1, _C, _RPROJ, _W), lambda t, r: (t, 0, r, 0))

    q, k, v = pl.pallas_call(
        _qkv_body,
        grid=(_T, _H // _RPROJ),
        in_specs=[img_spec, w_spec, b_spec, w_spec, b_spec, w_spec, b_spec],
        out_specs=[img_spec, img_spec, img_spec],
        out_shape=[jax.ShapeDtypeStruct((_T, _C, _H, _W), jnp.float32)] * 3,
        compiler_params=pltpu.CompilerParams(
            dimension_semantics=("parallel", "parallel")),
        interpret=interpret,
    )(vid4, Wq, col(bq), Wk, col(bk), Wv, col(bv))

    pad = ((0, 0), (0, 0), (_PAD, 2 * _WS - _PAD), (0, 0))
    kp = jnp.pad(k, pad, mode="reflect")
    vp = jnp.pad(v, pad, mode="reflect")

    blk = pl.BlockSpec((1, _HD, _RATT, _W), lambda t, h, r: (t, h, r, 0))
    blk_hi = pl.BlockSpec(
        (1, _HD, _WS, _W),
        lambda t, h, r: (t, h, (_RATT // _WS) * r + _RATT // _WS, 0))
    attn = pl.pallas_call(
        _attn_body,
        grid=(_T, _NH, _H // _RATT),
        in_specs=[blk, blk, blk_hi, blk, blk_hi],
        out_specs=blk,
        out_shape=jax.ShapeDtypeStruct((_T, _C, _H, _W), jnp.float32),
        compiler_params=pltpu.CompilerParams(
            dimension_semantics=("parallel", "parallel", "parallel")),
        interpret=interpret,
    )(q, kp, kp, vp, vp)

    out = pl.pallas_call(
        _out_body,
        grid=(_T, _H // _RPROJ),
        in_specs=[img_spec, w_spec, b_spec],
        out_specs=img_spec,
        out_shape=jax.ShapeDtypeStruct((_T, _C, _H, _W), jnp.float32),
        compiler_params=pltpu.CompilerParams(
            dimension_semantics=("parallel", "parallel")),
        interpret=interpret,
    )(attn, Wp, col(bp))

    return out[None]


def kernel(vid, Wq, bq, Wk, bk, Wv, bv, Wp, bp):
    return _forward(vid, Wq, bq, Wk, bk, Wv, bv, Wp, bp)
